# Initial kernel scaffold; baseline (speedup 1.0000x reference)
import jax
import jax.numpy as jnp
from jax.experimental import pallas as pl


def kernel(order_features, agv_features, waiting_features, buffer_features, order_agv_edge_feat, agv_waiting_edge_feat, agv_buffer_edge_feat, order_to_agv_edge_index, agv_to_order_edge_index, agv_to_waiting_edge_index, waiting_to_agv_edge_index, agv_to_buffer_edge_index, buffer_to_agv_edge_index, params):
    raise NotImplementedError("write your pallas kernel here")


# speedup vs baseline: 10.6677x; 10.6677x over previous
import jax
import jax.numpy as jnp
from jax.experimental import pallas as pl

_N = 50000
_ED = 64
_H = 4
_HD = 16
_L = 2
_NB = 5000
_EB = 4000


def _colstats_k(x_ref, s_ref, q_ref):
    @pl.when(pl.program_id(0) == 0)
    def _():
        s_ref[...] = jnp.zeros_like(s_ref)
        q_ref[...] = jnp.zeros_like(q_ref)
    x = x_ref[...]
    c = x.shape[1]
    s_ref[...] += jnp.broadcast_to(jnp.sum(x, axis=0, keepdims=True), (8, c))
    q_ref[...] += jnp.broadcast_to(jnp.sum(x * x, axis=0, keepdims=True), (8, c))


def _colstats(x, blk):
    r, c = x.shape
    nb = r // blk
    s, q = pl.pallas_call(
        _colstats_k,
        grid=(nb,),
        in_specs=[pl.BlockSpec((blk, c), lambda i: (i, 0))],
        out_specs=[pl.BlockSpec((8, c), lambda i: (0, 0)),
                   pl.BlockSpec((8, c), lambda i: (0, 0))],
        out_shape=[jax.ShapeDtypeStruct((8, c), jnp.float32),
                   jax.ShapeDtypeStruct((8, c), jnp.float32)],
    )(x)
    ssum = s[0]
    qsum = q[0]
    mean = ssum / r
    var = (qsum - r * mean * mean) / (r - 1)
    std = jnp.sqrt(var) + 1e-6
    return mean, std


def _prep_k(x_ref, sc_ref, sh_ref, w_ref, b_ref, o_ref):
    xn = x_ref[...] * sc_ref[...] + sh_ref[...]
    o_ref[...] = jnp.dot(xn, w_ref[...],
                         preferred_element_type=jnp.float32) + b_ref[...]


def _prep(x, scale, shift, Wt, bias):
    r, c = x.shape
    e = Wt.shape[1]
    nb = r // _NB
    return pl.pallas_call(
        _prep_k,
        grid=(nb,),
        in_specs=[pl.BlockSpec((_NB, c), lambda i: (i, 0)),
                  pl.BlockSpec((1, c), lambda i: (0, 0)),
                  pl.BlockSpec((1, c), lambda i: (0, 0)),
                  pl.BlockSpec((c, e), lambda i: (0, 0)),
                  pl.BlockSpec((1, e), lambda i: (0, 0))],
        out_specs=pl.BlockSpec((_NB, e), lambda i: (i, 0)),
        out_shape=jax.ShapeDtypeStruct((r, e), jnp.float32),
    )(x, scale.reshape(1, c), shift.reshape(1, c), Wt, bias.reshape(1, e))


def _proj_full_k(h_ref, w_ref, a_ref, hs_ref, s_ref):
    hs = jnp.dot(h_ref[...], w_ref[...], preferred_element_type=jnp.float32)
    hs_ref[...] = hs
    s_ref[...] = jnp.dot(hs, a_ref[...], preferred_element_type=jnp.float32)


def _proj_full(h, Wt, A):
    nb = _N // _NB
    return pl.pallas_call(
        _proj_full_k,
        grid=(nb,),
        in_specs=[pl.BlockSpec((_NB, _ED), lambda i: (i, 0)),
                  pl.BlockSpec((_ED, _ED), lambda i: (0, 0)),
                  pl.BlockSpec((_ED, _H), lambda i: (0, 0))],
        out_specs=[pl.BlockSpec((_NB, _ED), lambda i: (i, 0)),
                   pl.BlockSpec((_NB, _H), lambda i: (i, 0))],
        out_shape=[jax.ShapeDtypeStruct((_N, _ED), jnp.float32),
                   jax.ShapeDtypeStruct((_N, _H), jnp.float32)],
    )(h, Wt, A)


def _proj_s_k(h_ref, w_ref, a_ref, s_ref):
    hs = jnp.dot(h_ref[...], w_ref[...], preferred_element_type=jnp.float32)
    s_ref[...] = jnp.dot(hs, a_ref[...], preferred_element_type=jnp.float32)


def _proj_s(h, Wt, A):
    nb = _N // _NB
    return pl.pallas_call(
        _proj_s_k,
        grid=(nb,),
        in_specs=[pl.BlockSpec((_NB, _ED), lambda i: (i, 0)),
                  pl.BlockSpec((_ED, _ED), lambda i: (0, 0)),
                  pl.BlockSpec((_ED, _H), lambda i: (0, 0))],
        out_specs=pl.BlockSpec((_NB, _H), lambda i: (i, 0)),
        out_shape=jax.ShapeDtypeStruct((_N, _H), jnp.float32),
    )(h, Wt, A)


def _make_edge1(ed):
    def k(ssg_ref, sdg_ref, ef_ref, mt_ref, sc_ref, sh_ref, e_ref):
        efn = ef_ref[...] * sc_ref[...] + sh_ref[...]
        acc = ssg_ref[...] + sdg_ref[...]
        for d in range(ed):
            acc = acc + efn[:, d:d + 1] * mt_ref[d:d + 1, :]
        e_ref[...] = jnp.where(acc >= 0, acc, 0.2 * acc)
    return k


def _edge1(ssg, sdg, ef, MT, esc, esh):
    ne, ed = ef.shape
    nb = ne // _EB
    return pl.pallas_call(
        _make_edge1(ed),
        grid=(nb,),
        in_specs=[pl.BlockSpec((_EB, _H), lambda i: (i, 0)),
                  pl.BlockSpec((_EB, _H), lambda i: (i, 0)),
                  pl.BlockSpec((_EB, ed), lambda i: (i, 0)),
                  pl.BlockSpec((ed, _H), lambda i: (0, 0)),
                  pl.BlockSpec((1, ed), lambda i: (0, 0)),
                  pl.BlockSpec((1, ed), lambda i: (0, 0))],
        out_specs=pl.BlockSpec((_EB, _H), lambda i: (i, 0)),
        out_shape=jax.ShapeDtypeStruct((ne, _H), jnp.float32),
    )(ssg, sdg, ef, MT, esc.reshape(1, ed), esh.reshape(1, ed))


def _edge2_k(e_ref, mx_ref, o_ref):
    o_ref[...] = jnp.exp(e_ref[...] - mx_ref[...])


def _edge2(e, mxg):
    ne = e.shape[0]
    nb = ne // _EB
    return pl.pallas_call(
        _edge2_k,
        grid=(nb,),
        in_specs=[pl.BlockSpec((_EB, _H), lambda i: (i, 0)),
                  pl.BlockSpec((_EB, _H), lambda i: (i, 0))],
        out_specs=pl.BlockSpec((_EB, _H), lambda i: (i, 0)),
        out_shape=jax.ShapeDtypeStruct((ne, _H), jnp.float32),
    )(e, mxg)


def _edge3_k(p_ref, s_ref, hs_ref, o_ref):
    alpha = p_ref[...] / (s_ref[...] + 1e-8)
    ones = jnp.ones((1, _HD), jnp.float32)
    parts = [alpha[:, h:h + 1] * ones for h in range(_H)]
    o_ref[...] = jnp.concatenate(parts, axis=1) * hs_ref[...]


def _edge3(eexp, esg, hsg):
    ne = eexp.shape[0]
    nb = ne // _EB
    return pl.pallas_call(
        _edge3_k,
        grid=(nb,),
        in_specs=[pl.BlockSpec((_EB, _H), lambda i: (i, 0)),
                  pl.BlockSpec((_EB, _H), lambda i: (i, 0)),
                  pl.BlockSpec((_EB, _ED), lambda i: (i, 0))],
        out_specs=pl.BlockSpec((_EB, _ED), lambda i: (i, 0)),
        out_shape=jax.ShapeDtypeStruct((ne, _ED), jnp.float32),
    )(eexp, esg, hsg)


def _make_ln(nm):
    def k(*refs):
        h = refs[0][...]
        for i in range(nm):
            h = h + refs[1 + i][...]
        g = refs[1 + nm][...]
        b = refs[2 + nm][...]
        mu = jnp.mean(h, axis=1, keepdims=True)
        var = jnp.mean((h - mu) ** 2, axis=1, keepdims=True)
        y = (h - mu) / jnp.sqrt(var + 1e-5) * g + b
        refs[3 + nm][...] = y
        s_ref = refs[4 + nm]
        @pl.when(pl.program_id(0) == 0)
        def _():
            s_ref[...] = jnp.zeros_like(s_ref)
        s_ref[...] += jnp.broadcast_to(
            jnp.sum(y, axis=0, keepdims=True), (8, _ED))
    return k


def _ln(h, msgs, gamma, beta):
    nb = _N // _NB
    nm = len(msgs)
    row = pl.BlockSpec((_NB, _ED), lambda i: (i, 0))
    out, part = pl.pallas_call(
        _make_ln(nm),
        grid=(nb,),
        in_specs=[row] * (1 + nm) + [pl.BlockSpec((1, _ED), lambda i: (0, 0))] * 2,
        out_specs=[row, pl.BlockSpec((8, _ED), lambda i: (0, 0))],
        out_shape=[jax.ShapeDtypeStruct((_N, _ED), jnp.float32),
                   jax.ShapeDtypeStruct((8, _ED), jnp.float32)],
    )(h, *msgs, gamma.reshape(1, _ED), beta.reshape(1, _ED))
    return out, part


def _mlp_k(g_ref, w1_ref, b1_ref, w2_ref, b2_ref, o_ref):
    g1 = jnp.dot(g_ref[...], w1_ref[...],
                 preferred_element_type=jnp.float32) + b1_ref[...]
    g1 = jnp.maximum(g1, 0.0)
    o_ref[...] = jnp.dot(g1, w2_ref[...],
                         preferred_element_type=jnp.float32) + b2_ref[...]


def _mlp(gvec, W1t, b1, W2t, b2):
    return pl.pallas_call(
        _mlp_k,
        out_shape=jax.ShapeDtypeStruct((1, _ED), jnp.float32),
    )(gvec, W1t, b1.reshape(1, _ED), W2t, b2.reshape(1, _ED))


def _gat_rel(Pl, h_src, h_dst, ei, ef, esc, esh):
    ed = ef.shape[1]
    eye = jnp.eye(_H, dtype=jnp.float32)
    A_src = (eye[:, None, :] * Pl['a_src'][:, :, None]).reshape(_ED, _H)
    A_dst = (eye[:, None, :] * Pl['a_dst'][:, :, None]).reshape(_ED, _H)
    MT = (Pl['W_edge'].reshape(_H, _HD, ed) * Pl['a_edge'][:, :, None]).sum(1).T
    src, dst = ei[0], ei[1]
    hs, ssrc = _proj_full(h_src, Pl['W_src'].T, A_src)
    sdst = _proj_s(h_dst, Pl['W_dst'].T, A_dst)
    e = _edge1(ssrc[src], sdst[dst], ef, MT, esc, esh)
    emax = jax.ops.segment_max(e, dst, num_segments=_N)
    emax = jnp.where(jnp.isfinite(emax), emax, 0.0)
    eexp = _edge2(e, emax[dst])
    esum = jax.ops.segment_sum(eexp, dst, num_segments=_N)
    msg = _edge3(eexp, esum[dst], hs[src])
    return jax.ops.segment_sum(msg, dst, num_segments=_N)


def kernel(order_features, agv_features, waiting_features, buffer_features,
           order_agv_edge_feat, agv_waiting_edge_feat, agv_buffer_edge_feat,
           order_to_agv_edge_index, agv_to_order_edge_index,
           agv_to_waiting_edge_index, waiting_to_agv_edge_index,
           agv_to_buffer_edge_index, buffer_to_agv_edge_index, params):
    P = params
    h = {}
    feats = {'order': order_features, 'agv': agv_features,
             'waiting': waiting_features, 'buffer': buffer_features}
    for t, x in feats.items():
        mean, std = _colstats(x, _NB)
        g = P[t + '_std_gamma']
        b = P[t + '_std_beta']
        scale = g / std
        shift = b - mean * scale
        h[t] = _prep(x, scale, shift, P[t + '_embed_W'].T, P[t + '_embed_b'])

    estats = {}
    for en, ef in [('oa', order_agv_edge_feat), ('aw', agv_waiting_edge_feat),
                   ('ab', agv_buffer_edge_feat)]:
        mean, std = _colstats(ef, _EB)
        g = P['edge_' + en + '_std_gamma']
        b = P['edge_' + en + '_std_beta']
        scale = g / std
        estats[en] = (scale, b - mean * scale)

    parts = {}
    for l in range(_L):
        def rp(rel):
            return {k: P[rel + str(l) + '_' + k]
                    for k in ['W_src', 'W_dst', 'W_edge',
                              'a_src', 'a_dst', 'a_edge']}
        esc_oa, esh_oa = estats['oa']
        esc_aw, esh_aw = estats['aw']
        esc_ab, esh_ab = estats['ab']
        m_a1 = _gat_rel(rp('o2a'), h['order'], h['agv'],
                        order_to_agv_edge_index, order_agv_edge_feat,
                        esc_oa, esh_oa)
        m_o = _gat_rel(rp('a2o'), h['agv'], h['order'],
                       agv_to_order_edge_index, order_agv_edge_feat,
                       esc_oa, esh_oa)
        m_w = _gat_rel(rp('a2w'), h['agv'], h['waiting'],
                       agv_to_waiting_edge_index, agv_waiting_edge_feat,
                       esc_aw, esh_aw)
        m_a2 = _gat_rel(rp('w2a'), h['waiting'], h['agv'],
                        waiting_to_agv_edge_index, agv_waiting_edge_feat,
                        esc_aw, esh_aw)
        m_b = _gat_rel(rp('a2b'), h['agv'], h['buffer'],
                       agv_to_buffer_edge_index, agv_buffer_edge_feat,
                       esc_ab, esh_ab)
        m_a3 = _gat_rel(rp('b2a'), h['buffer'], h['agv'],
                        buffer_to_agv_edge_index, agv_buffer_edge_feat,
                        esc_ab, esh_ab)
        sfx = str(l)
        h['order'], parts['order'] = _ln(h['order'], [m_o],
                                         P['order_ln' + sfx + '_gamma'],
                                         P['order_ln' + sfx + '_beta'])
        h['agv'], parts['agv'] = _ln(h['agv'], [m_a1, m_a2, m_a3],
                                     P['agv_ln' + sfx + '_gamma'],
                                     P['agv_ln' + sfx + '_beta'])
        h['waiting'], parts['waiting'] = _ln(h['waiting'], [m_w],
                                             P['waiting_ln' + sfx + '_gamma'],
                                             P['waiting_ln' + sfx + '_beta'])
        h['buffer'], parts['buffer'] = _ln(h['buffer'], [m_b],
                                           P['buffer_ln' + sfx + '_gamma'],
                                           P['buffer_ln' + sfx + '_beta'])

    gvec = jnp.concatenate([parts[t][0] / _N
                            for t in ['order', 'agv', 'waiting', 'buffer']])
    g = _mlp(gvec.reshape(1, 4 * _ED), P['glob_W1'].T, P['glob_b1'],
             P['glob_W2'].T, P['glob_b2'])
    return (h['order'], h['agv'], h['waiting'], h['buffer'], g.reshape(_ED))
